# baseline (speedup 1.0000x reference)
import functools

import jax
import jax.numpy as jnp
from jax import lax
from jax.experimental import pallas as pl
from jax.experimental.pallas import tpu as pltpu
from jax.experimental.pallas import tpu_sc as plsc

VOCAB = 100000
EMB = 64
BATCH = 1024
CTX = 20

NUM_CORES = 2
NUM_SUBCORES = 16
NUM_WORKERS = NUM_CORES * NUM_SUBCORES
BPW = BATCH // NUM_WORKERS
IPW = BPW * CTX

LANES = 16
VB = 2048


def _pooled_sc(idx_flat, emb_table):
    mesh = plsc.VectorSubcoreMesh(core_axis_name="c", subcore_axis_name="s")

    @functools.partial(
        pl.kernel,
        mesh=mesh,
        out_type=jax.ShapeDtypeStruct((BATCH, EMB), jnp.float32),
        scratch_types=[
            pltpu.VMEM((IPW,), jnp.int32),
            pltpu.VMEM((IPW, EMB), jnp.float32),
            pltpu.VMEM((BPW, EMB), jnp.float32),
            pltpu.SemaphoreType.DMA,
        ],
        compiler_params=pltpu.CompilerParams(use_tc_tiling_on_sc=False),
    )
    def k(table_hbm, idx_hbm, out_hbm, idx_v, rows_v, pooled_v, sem):
        wid = lax.axis_index("s") * NUM_CORES + lax.axis_index("c")
        pltpu.sync_copy(idx_hbm.at[pl.ds(wid * IPW, IPW)], idx_v)
        pltpu.async_copy(table_hbm.at[idx_v], rows_v, sem).wait()

        def batch_body(b, carry):
            base = b * CTX
            for d in range(EMB // LANES):
                acc = rows_v[base, pl.ds(d * LANES, LANES)]

                def row_body(j, a):
                    return a + rows_v[base + j, pl.ds(d * LANES, LANES)]

                acc = lax.fori_loop(1, CTX, row_body, acc)
                pooled_v[b, pl.ds(d * LANES, LANES)] = acc
            return carry

        lax.fori_loop(0, BPW, batch_body, 0)
        pltpu.sync_copy(pooled_v, out_hbm.at[pl.ds(wid * BPW, BPW)])

    return k(emb_table, idx_flat)


NSTEPS = 49
TAIL = VOCAB - (NSTEPS - 1) * VB
NBUF = 4


def _mm_body(p_ref, wt_ref, o_hbm, acc, sems):
    i = pl.program_id(0)
    slot = lax.rem(i, NBUF)

    for k in range(NBUF):
        @pl.when(jnp.logical_and(i >= NBUF, slot == k))
        def _(k=k):
            pltpu.make_async_copy(
                acc.at[k],
                o_hbm.at[pl.ds((i - NBUF) * VB, VB)],
                sems.at[k],
            ).wait()

    acc[slot] = lax.dot_general(
        wt_ref[...],
        p_ref[...],
        dimension_numbers=(((0,), (1,)), ((), ())),
        preferred_element_type=jnp.float32,
    )

    for k in range(NBUF):
        @pl.when(jnp.logical_and(i < NSTEPS - 1, slot == k))
        def _(k=k):
            pltpu.make_async_copy(
                acc.at[k], o_hbm.at[pl.ds(i * VB, VB)], sems.at[k]
            ).start()

    @pl.when(i == NSTEPS - 1)
    def _():
        last = NSTEPS - 1
        pltpu.make_async_copy(
            acc.at[last % NBUF, pl.ds(0, TAIL)],
            o_hbm.at[pl.ds(last * VB, TAIL)],
            sems.at[last % NBUF],
        ).start()
        for j in range(NSTEPS - NBUF, NSTEPS - 1):
            pltpu.make_async_copy(
                acc.at[j % NBUF],
                o_hbm.at[pl.ds(j * VB, VB)],
                sems.at[j % NBUF],
            ).wait()
        pltpu.make_async_copy(
            acc.at[last % NBUF, pl.ds(0, TAIL)],
            o_hbm.at[pl.ds(last * VB, TAIL)],
            sems.at[last % NBUF],
        ).wait()


def kernel(x, emb_table, W):
    idx_flat = x.reshape(-1).astype(jnp.int32)
    pooled = _pooled_sc(idx_flat, emb_table)
    wt = W.T
    out_t = pl.pallas_call(
        _mm_body,
        grid=(NSTEPS,),
        in_specs=[
            pl.BlockSpec((BATCH, EMB), lambda i: (0, 0)),
            pl.BlockSpec((EMB, VB), lambda i: (0, i)),
        ],
        out_specs=pl.BlockSpec(memory_space=pl.ANY),
        out_shape=jax.ShapeDtypeStruct((VOCAB, BATCH), jnp.float32),
        scratch_shapes=[
            pltpu.VMEM((NBUF, VB, BATCH), jnp.float32),
            pltpu.SemaphoreType.DMA((NBUF,)),
        ],
        compiler_params=pltpu.CompilerParams(
            dimension_semantics=("arbitrary",),
        ),
    )(pooled, wt)
    return out_t.T


# speedup vs baseline: 1.2967x; 1.2967x over previous
import functools

import jax
import jax.numpy as jnp
from jax import lax
from jax.experimental import pallas as pl
from jax.experimental.pallas import tpu as pltpu
from jax.experimental.pallas import tpu_sc as plsc

VOCAB = 100000
EMB = 64
BATCH = 1024
CTX = 20

NUM_CORES = 2
NUM_SUBCORES = 16
NUM_WORKERS = NUM_CORES * NUM_SUBCORES
BPW = BATCH // NUM_WORKERS
IPW = BPW * CTX

LANES = 16
VB = 2048


PLANES_PER_WORKER = EMB // NUM_WORKERS


def _pooled_sc(idx_t, emb_t):
    mesh = plsc.VectorSubcoreMesh(core_axis_name="c", subcore_axis_name="s")

    @functools.partial(
        pl.kernel,
        mesh=mesh,
        out_type=jax.ShapeDtypeStruct((EMB, BATCH), jnp.float32),
        scratch_types=[
            pltpu.VMEM((VOCAB,), jnp.float32),
            pltpu.VMEM((CTX, BATCH), jnp.int32),
            pltpu.VMEM((PLANES_PER_WORKER, BATCH), jnp.float32),
            pltpu.SemaphoreType.DMA,
        ],
        compiler_params=pltpu.CompilerParams(
            use_tc_tiling_on_sc=True, needs_layout_passes=False
        ),
    )
    def k(emb_hbm, idx_hbm, out_hbm, plane_v, idx_v, pool_v, sem):
        wid = lax.axis_index("s") * NUM_CORES + lax.axis_index("c")
        pltpu.sync_copy(idx_hbm, idx_v)
        for r in range(PLANES_PER_WORKER):
            kplane = wid * PLANES_PER_WORKER + r
            pltpu.sync_copy(emb_hbm.at[kplane], plane_v)

            def group_body(g, carry, r=r):
                acc = jnp.zeros((LANES,), jnp.float32)
                for c in range(CTX):
                    idxs = idx_v[c, pl.ds(g * LANES, LANES)]
                    acc = acc + plsc.load_gather(plane_v, [idxs])
                pool_v[r, pl.ds(g * LANES, LANES)] = acc
                return carry

            lax.fori_loop(0, BATCH // LANES, group_body, 0)
        pltpu.sync_copy(
            pool_v, out_hbm.at[pl.ds(wid * PLANES_PER_WORKER, PLANES_PER_WORKER)]
        )

    return k(emb_t, idx_t)


NSTEPS = 49
TAIL = VOCAB - (NSTEPS - 1) * VB
NBUF = 4


def _mm_body(p_ref, wt_ref, o_hbm, acc, sems):
    i = pl.program_id(0)
    slot = lax.rem(i, NBUF)

    for k in range(NBUF):
        @pl.when(jnp.logical_and(i >= NBUF, slot == k))
        def _(k=k):
            pltpu.make_async_copy(
                acc.at[k],
                o_hbm.at[pl.ds((i - NBUF) * VB, VB)],
                sems.at[k],
            ).wait()

    acc[slot] = lax.dot_general(
        wt_ref[...],
        p_ref[...],
        dimension_numbers=(((0,), (0,)), ((), ())),
        preferred_element_type=jnp.float32,
    )

    for k in range(NBUF):
        @pl.when(jnp.logical_and(i < NSTEPS - 1, slot == k))
        def _(k=k):
            pltpu.make_async_copy(
                acc.at[k], o_hbm.at[pl.ds(i * VB, VB)], sems.at[k]
            ).start()

    @pl.when(i == NSTEPS - 1)
    def _():
        last = NSTEPS - 1
        pltpu.make_async_copy(
            acc.at[last % NBUF, pl.ds(0, TAIL)],
            o_hbm.at[pl.ds(last * VB, TAIL)],
            sems.at[last % NBUF],
        ).start()
        for j in range(NSTEPS - NBUF, NSTEPS - 1):
            pltpu.make_async_copy(
                acc.at[j % NBUF],
                o_hbm.at[pl.ds(j * VB, VB)],
                sems.at[j % NBUF],
            ).wait()
        pltpu.make_async_copy(
            acc.at[last % NBUF, pl.ds(0, TAIL)],
            o_hbm.at[pl.ds(last * VB, TAIL)],
            sems.at[last % NBUF],
        ).wait()


def kernel(x, emb_table, W):
    idx_t = x.T.astype(jnp.int32)
    emb_t = emb_table.T
    pooled_t = _pooled_sc(idx_t, emb_t)
    wt = W.T
    out_t = pl.pallas_call(
        _mm_body,
        grid=(NSTEPS,),
        in_specs=[
            pl.BlockSpec((EMB, BATCH), lambda i: (0, 0)),
            pl.BlockSpec((EMB, VB), lambda i: (0, i)),
        ],
        out_specs=pl.BlockSpec(memory_space=pl.ANY),
        out_shape=jax.ShapeDtypeStruct((VOCAB, BATCH), jnp.float32),
        scratch_shapes=[
            pltpu.VMEM((NBUF, VB, BATCH), jnp.float32),
            pltpu.SemaphoreType.DMA((NBUF,)),
        ],
        compiler_params=pltpu.CompilerParams(
            dimension_semantics=("arbitrary",),
        ),
    )(pooled_t, wt)
    return out_t.T
